# baseline (speedup 1.0000x reference)
import functools

import jax
import jax.numpy as jnp
from jax import lax
from jax.experimental import pallas as pl
from jax.experimental.pallas import tpu as pltpu
from jax.experimental.pallas import tpu_sc as plsc


def _make_pooling_part(B, CTX, D, V):
    info = plsc.get_sparse_core_info()
    nc, ns = info.num_cores, info.num_subcores
    nw = nc * ns
    n_slabs = D // 8
    nq = nw // n_slabs
    n_idx = B * CTX
    lanes = 128
    n_wins = 4 * nq
    half_w = (-(-V // (lanes * n_wins))) * lanes
    wins = []
    for wg in range(n_wins):
        lo = wg * half_w
        hi = min(lo + half_w, V)
        main = (hi - lo) // lanes * lanes
        wins.append((lo, main, hi - lo - main))

    mesh = plsc.VectorSubcoreMesh(core_axis_name="c", subcore_axis_name="s")

    @functools.partial(
        pl.kernel,
        mesh=mesh,
        out_type=jax.ShapeDtypeStruct((nq, D, B), jnp.float32),
        compiler_params=pltpu.CompilerParams(needs_layout_passes=False),
        scratch_types=[
            pltpu.VMEM((n_idx,), jnp.int32),
            pltpu.VMEM((8, half_w), jnp.float32),
            pltpu.VMEM((8, half_w), jnp.float32),
            pltpu.VMEM((8, B), jnp.float32),
            pltpu.SemaphoreType.DMA,
            pltpu.SemaphoreType.DMA,
        ],
    )
    def pool(idx_hbm, table_hbm, tail_hbm, out_hbm, idx_v, slab_a, slab_b, acc_v, sem_a, sem_b):
        wid = lax.axis_index("s") * nc + lax.axis_index("c")
        q = wid % nq
        s = wid // nq
        pltpu.sync_copy(idx_hbm, idx_v)

        zero = jnp.zeros((16,), jnp.float32)

        def zbody(c, carry):
            base = c * 16
            for di in range(8):
                acc_v[di, pl.ds(base, 16)] = zero
            return carry

        lax.fori_loop(0, B // 16, zbody, 0)

        def gather_chunk(lo, width, ref, dbase):
            def body(c, carry):
                base = c * 16
                ivs = []
                for k in range(CTX):
                    iv = idx_v[pl.ds(k * B + base, 16)]
                    loc = iv - lo
                    m = (loc >= 0) & (loc < width)
                    loc = jnp.where(m, loc, 0)
                    ivs.append((loc, m))
                for di in range(8):
                    dvec = jnp.full((16,), di, jnp.int32) + dbase
                    acc = acc_v[di, pl.ds(base, 16)]
                    for loc, m in ivs:
                        g = plsc.load_gather(ref, [dvec, loc], mask=m)
                        acc = acc + jnp.where(m, g, 0.0)
                    acc_v[di, pl.ds(base, 16)] = acc
                return carry

            lax.fori_loop(0, B // 16, body, 0)

        bufs = [slab_a, slab_b]
        sems = [sem_a, sem_b]

        for qq in range(nq):

            @pl.when(q == qq)
            def _():
                my = [wins[qq * 4 + h] for h in range(4)]

                def start_dma(i):
                    lo, main, tail = my[i]
                    cps = [
                        pltpu.async_copy(
                            table_hbm.at[pl.ds(s * 8, 8), pl.ds(lo, main)],
                            bufs[i % 2].at[pl.ds(0, 8), pl.ds(0, main)],
                            sems[i % 2],
                        )
                    ]
                    if tail:
                        cps.append(
                            pltpu.async_copy(
                                tail_hbm.at[pl.ds(s * 8, 8)],
                                bufs[i % 2].at[pl.ds(0, 8), pl.ds(main, 128)],
                                sems[i % 2],
                            )
                        )
                    return cps

                pending = start_dma(0)
                for i in range(4):
                    for cp in pending:
                        cp.wait()
                    if i + 1 < 4:
                        pending = start_dma(i + 1)
                    lo, main, tail = my[i]
                    gather_chunk(lo, main + tail, bufs[i % 2], 0)

        pltpu.sync_copy(acc_v, out_hbm.at[q, pl.ds(s * 8, 8)])

    def run(inputs, emb_table):
        idx = inputs.T.reshape(n_idx)
        emb_t = emb_table.T
        last_tail = wins[-1][2]
        tail_t = jnp.pad(
            emb_table[V - last_tail :].T, ((0, 0), (0, 128 - last_tail))
        )
        return pool(idx, emb_t, tail_t)

    return run, nq


def _project(part, W, b, vb=4096):
    nq, D, B = part.shape
    V = W.shape[0]
    w_t = W.T
    b2 = b.reshape(1, V)

    def mm(part_ref, w_ref, b_ref, o_ref):
        p = part_ref[0]
        for qq in range(1, nq):
            p = p + part_ref[qq]
        o_ref[...] = lax.dot_general(
            w_ref[...],
            p,
            dimension_numbers=(((0,), (0,)), ((), ())),
            preferred_element_type=jnp.float32,
        ) + jnp.transpose(b_ref[...])

    out_t = pl.pallas_call(
        mm,
        grid=(pl.cdiv(V, vb),),
        in_specs=[
            pl.BlockSpec((nq, D, B), lambda v: (0, 0, 0)),
            pl.BlockSpec((D, vb), lambda v: (0, v)),
            pl.BlockSpec((1, vb), lambda v: (0, v)),
        ],
        out_specs=pl.BlockSpec((vb, B), lambda v: (v, 0)),
        out_shape=jax.ShapeDtypeStruct((V, B), jnp.float32),
    )(part, w_t, b2)
    return out_t.T


def kernel(inputs, emb_table, W, b):
    B, CTX = inputs.shape
    V, D = emb_table.shape
    run, _ = _make_pooling_part(B, CTX, D, V)
    part = run(inputs, emb_table)
    return _project(part, W, b)


# speedup vs baseline: 1.0498x; 1.0498x over previous
import functools

import jax
import jax.numpy as jnp
from jax import lax
from jax.experimental import pallas as pl
from jax.experimental.pallas import tpu as pltpu
from jax.experimental.pallas import tpu_sc as plsc


def _make_pooling_part(B, CTX, D, V):
    info = plsc.get_sparse_core_info()
    nc, ns = info.num_cores, info.num_subcores
    nw = nc * ns
    n_slabs = D // 8
    nq = nw // n_slabs
    n_idx = B * CTX
    lanes = 128
    n_wins = 2 * nq
    full_w = (-(-V // (lanes * n_wins))) * lanes
    wins = []
    for wg in range(n_wins):
        lo = wg * full_w
        hi = min(lo + full_w, V)
        main = (hi - lo) // lanes * lanes
        wins.append((lo, main, hi - lo - main))

    mesh = plsc.VectorSubcoreMesh(core_axis_name="c", subcore_axis_name="s")

    @functools.partial(
        pl.kernel,
        mesh=mesh,
        out_type=jax.ShapeDtypeStruct((nq, D, B), jnp.float32),
        compiler_params=pltpu.CompilerParams(needs_layout_passes=False),
        scratch_types=[
            pltpu.VMEM((n_idx,), jnp.int32),
            pltpu.VMEM((8, full_w), jnp.float32),
            pltpu.VMEM((8, B), jnp.float32),
        ],
    )
    def pool(idx_hbm, table_hbm, tail_hbm, out_hbm, idx_v, slab_v, acc_v):
        wid = lax.axis_index("s") * nc + lax.axis_index("c")
        q = wid % nq
        s = wid // nq
        pltpu.sync_copy(idx_hbm, idx_v)

        zero = jnp.zeros((16,), jnp.float32)

        def zbody(c, carry):
            base = c * 16
            for di in range(8):
                acc_v[di, pl.ds(base, 16)] = zero
            return carry

        lax.fori_loop(0, B // 16, zbody, 0)

        def gather_chunk(lo, width, ref, dbase):
            def body(c, carry):
                base = c * 16
                ivs = []
                for k in range(CTX):
                    iv = idx_v[pl.ds(k * B + base, 16)]
                    loc = iv - lo
                    m = (loc >= 0) & (loc < width)
                    loc = jnp.where(m, loc, 0)
                    ivs.append((loc, m))
                for di in range(8):
                    dvec = jnp.full((16,), di, jnp.int32) + dbase
                    acc = acc_v[di, pl.ds(base, 16)]
                    for loc, m in ivs:
                        g = plsc.load_gather(ref, [dvec, loc], mask=m)
                        acc = acc + jnp.where(m, g, 0.0)
                    acc_v[di, pl.ds(base, 16)] = acc
                return carry

            lax.fori_loop(0, B // 16, body, 0)

        for cid in range(n_wins):
            lo, main, tail = wins[cid]

            @pl.when(q == cid // 2)
            def _():
                pltpu.sync_copy(
                    table_hbm.at[pl.ds(s * 8, 8), pl.ds(lo, main)],
                    slab_v.at[pl.ds(0, 8), pl.ds(0, main)],
                )
                if tail:
                    pltpu.sync_copy(
                        tail_hbm.at[pl.ds(s * 8, 8)],
                        slab_v.at[pl.ds(0, 8), pl.ds(main, 128)],
                    )
                gather_chunk(lo, main + tail, slab_v, 0)

        pltpu.sync_copy(acc_v, out_hbm.at[q, pl.ds(s * 8, 8)])

    def run(inputs, emb_table):
        idx = inputs.T.reshape(n_idx)
        emb_t = emb_table.T
        last_tail = wins[-1][2]
        tail_t = jnp.pad(
            emb_table[V - last_tail :].T, ((0, 0), (0, 128 - last_tail))
        )
        return pool(idx, emb_t, tail_t)

    return run, nq


def _project(part, W, b, vb=4096):
    nq, D, B = part.shape
    V = W.shape[0]
    w_t = W.T
    b2 = b.reshape(1, V)

    def mm(part_ref, w_ref, b_ref, o_ref):
        p = part_ref[0]
        for qq in range(1, nq):
            p = p + part_ref[qq]
        o_ref[...] = lax.dot_general(
            w_ref[...],
            p,
            dimension_numbers=(((0,), (0,)), ((), ())),
            preferred_element_type=jnp.float32,
        ) + jnp.transpose(b_ref[...])

    out_t = pl.pallas_call(
        mm,
        grid=(pl.cdiv(V, vb),),
        in_specs=[
            pl.BlockSpec((nq, D, B), lambda v: (0, 0, 0)),
            pl.BlockSpec((D, vb), lambda v: (0, v)),
            pl.BlockSpec((1, vb), lambda v: (0, v)),
        ],
        out_specs=pl.BlockSpec((vb, B), lambda v: (v, 0)),
        out_shape=jax.ShapeDtypeStruct((V, B), jnp.float32),
    )(part, w_t, b2)
    return out_t.T


def kernel(inputs, emb_table, W, b):
    B, CTX = inputs.shape
    V, D = emb_table.shape
    run, _ = _make_pooling_part(B, CTX, D, V)
    part = run(inputs, emb_table)
    return _project(part, W, b)
